# baseline (speedup 1.0000x reference)
import functools

import jax
import jax.numpy as jnp
from jax import lax
from jax.experimental import pallas as pl
from jax.experimental.pallas import tpu as pltpu
from jax.experimental.pallas import tpu_sc as plsc

N = 10000
E = 320000
D = 128
DE = 16
H = 128
OUT = 4
EPS = 1e-5

L = 16
NC = 2
NS = 16
NW = NC * NS
EPW = E // NW
CB = 80
NCHUNK = EPW // CB
RPT = 632
NP = RPT * NS

BN = 2000
BE = 4000


def _node_proj_body(h_ref, w_ref, b_ref, a_ref, b_out_ref):
    h = h_ref[...]
    a_ref[...] = jnp.dot(h, w_ref[0:D, :], preferred_element_type=jnp.float32) + b_ref[...]
    b_out_ref[...] = jnp.dot(h, w_ref[D:2 * D, :], preferred_element_type=jnp.float32)


def _node_proj(h, W, b):
    return pl.pallas_call(
        _node_proj_body,
        grid=(N // BN,),
        in_specs=[
            pl.BlockSpec((BN, D), lambda i: (i, 0)),
            pl.BlockSpec((2 * D + DE, H), lambda i: (0, 0)),
            pl.BlockSpec((1, H), lambda i: (0, 0)),
        ],
        out_specs=[
            pl.BlockSpec((BN, H), lambda i: (i, 0)),
            pl.BlockSpec((BN, H), lambda i: (i, 0)),
        ],
        out_shape=[jax.ShapeDtypeStruct((N, H), jnp.float32)] * 2,
    )(h, W, b.reshape(1, H))


def _edge_proj_body(ea_ref, w_ref, o_ref):
    o_ref[...] = jnp.dot(ea_ref[...], w_ref[...], preferred_element_type=jnp.float32)


def _edge_proj(ea, We):
    return pl.pallas_call(
        _edge_proj_body,
        grid=(E // BE,),
        in_specs=[
            pl.BlockSpec((BE, DE), lambda i: (i, 0)),
            pl.BlockSpec((DE, H), lambda i: (0, 0)),
        ],
        out_specs=pl.BlockSpec((BE, H), lambda i: (i, 0)),
        out_shape=jax.ShapeDtypeStruct((E, H), jnp.float32),
    )(ea, We)


def _mean_bn_relu(sp_ref, cnt_ref, wb_ref, bb_ref, g_ref, be_ref):
    S = sp_ref[0, 0:N, :] + sp_ref[1, 0:N, :]
    cnt = cnt_ref[0, 0:N, 0:1] + cnt_ref[1, 0:N, 0:1]
    m = jnp.dot(S, wb_ref[...], preferred_element_type=jnp.float32)
    m = (m + cnt * bb_ref[...]) / jnp.maximum(cnt, 1.0)
    mu = jnp.mean(m, axis=0, keepdims=True)
    var = jnp.mean((m - mu) ** 2, axis=0, keepdims=True)
    return jnp.maximum((m - mu) * lax.rsqrt(var + EPS) * g_ref[...] + be_ref[...], 0.0)


def _post_body(sp_ref, cnt_ref, wb_ref, bb_ref, g_ref, be_ref, h_ref):
    h_ref[...] = _mean_bn_relu(sp_ref, cnt_ref, wb_ref, bb_ref, g_ref, be_ref)


def _post(Sp, cnt, Wb, bb, g, be):
    return pl.pallas_call(
        _post_body,
        out_shape=jax.ShapeDtypeStruct((N, H), jnp.float32),
    )(Sp, cnt, Wb, bb.reshape(1, H), g.reshape(1, H), be.reshape(1, H))


def _post_final_body(sp_ref, cnt_ref, wb_ref, bb_ref, g_ref, be_ref,
                     wo_ref, bo_ref, o_ref):
    h = _mean_bn_relu(sp_ref, cnt_ref, wb_ref, bb_ref, g_ref, be_ref)
    o_ref[...] = jnp.dot(h, wo_ref[...], preferred_element_type=jnp.float32) + bo_ref[...]


def _post_final(Sp, cnt, Wb, bb, g, be, Wo, bo):
    return pl.pallas_call(
        _post_final_body,
        out_shape=jax.ShapeDtypeStruct((N, OUT), jnp.float32),
    )(Sp, cnt, Wb, bb.reshape(1, H), g.reshape(1, H), be.reshape(1, H),
      Wo, bo.reshape(1, OUT))


def _sc_edge_body(a_hbm, b_hbm, ep_hbm, dst_hbm, src_hbm, z_hbm, out_hbm,
                  dsti, srci, arows, brows, eprov, msg, S, sema, semb):
    cid = lax.axis_index("c")
    sid = lax.axis_index("s")
    wid = sid * NC + cid
    base = wid * EPW

    pltpu.sync_copy(z_hbm.at[pl.ds(sid * RPT, RPT)], S.at[pl.ds(sid * RPT, RPT)])
    plsc.subcore_barrier()

    def chunk(i, c):
        off = base + i * CB
        pltpu.sync_copy(dst_hbm.at[pl.ds(off, CB)], dsti)
        pltpu.sync_copy(src_hbm.at[pl.ds(off, CB)], srci)
        ga = pltpu.async_copy(a_hbm.at[dsti], arows, sema)
        gb = pltpu.async_copy(b_hbm.at[srci], brows, semb)
        pltpu.sync_copy(ep_hbm.at[pl.ds(off, CB)], eprov)
        ga.wait()
        gb.wait()

        def row(r, cc):
            for k in range(H // L):
                s = k * L
                v = arows[r, pl.ds(s, L)] + brows[r, pl.ds(s, L)] + eprov[r, pl.ds(s, L)]
                msg[r, pl.ds(s, L)] = jnp.maximum(v, 0.0)
            return cc

        lax.fori_loop(0, CB, row, 0, unroll=2)
        pltpu.sync_copy(msg, S.at[dsti], add=True)
        return c

    lax.fori_loop(0, NCHUNK, chunk, 0)
    plsc.subcore_barrier()

    pltpu.sync_copy(S.at[pl.ds(sid * RPT, RPT)],
                    out_hbm.at[cid, pl.ds(sid * RPT, RPT)])


_sc_edge = functools.partial(
    pl.kernel,
    mesh=plsc.VectorSubcoreMesh(core_axis_name="c", subcore_axis_name="s",
                                num_cores=NC, num_subcores=NS),
    out_type=jax.ShapeDtypeStruct((NC, NP, H), jnp.float32),
    scratch_types=[
        pltpu.VMEM((CB,), jnp.int32),
        pltpu.VMEM((CB,), jnp.int32),
        pltpu.VMEM((CB, H), jnp.float32),
        pltpu.VMEM((CB, H), jnp.float32),
        pltpu.VMEM((CB, H), jnp.float32),
        pltpu.VMEM((CB, H), jnp.float32),
        pltpu.VMEM_SHARED((NP, H), jnp.float32),
        pltpu.SemaphoreType.DMA,
        pltpu.SemaphoreType.DMA,
    ],
)(_sc_edge_body)


def _sc_count_body(dst_hbm, z_hbm, ones_hbm, out_hbm, dsti, ones_v, S):
    cid = lax.axis_index("c")
    sid = lax.axis_index("s")
    wid = sid * NC + cid

    pltpu.sync_copy(z_hbm.at[pl.ds(sid * RPT, RPT)], S.at[pl.ds(sid * RPT, RPT)])
    pltpu.sync_copy(ones_hbm, ones_v)
    plsc.subcore_barrier()

    base = wid * EPW

    def chunk(i, c):
        off = base + i * CB
        pltpu.sync_copy(dst_hbm.at[pl.ds(off, CB)], dsti)
        pltpu.sync_copy(ones_v, S.at[dsti], add=True)
        return c

    lax.fori_loop(0, NCHUNK, chunk, 0)
    plsc.subcore_barrier()

    pltpu.sync_copy(S.at[pl.ds(sid * RPT, RPT)],
                    out_hbm.at[cid, pl.ds(sid * RPT, RPT)])


_sc_count = functools.partial(
    pl.kernel,
    mesh=plsc.VectorSubcoreMesh(core_axis_name="c", subcore_axis_name="s",
                                num_cores=NC, num_subcores=NS),
    out_type=jax.ShapeDtypeStruct((NC, NP, H), jnp.float32),
    scratch_types=[
        pltpu.VMEM((CB,), jnp.int32),
        pltpu.VMEM((CB, H), jnp.float32),
        pltpu.VMEM_SHARED((NP, H), jnp.float32),
    ],
)(_sc_count_body)


def kernel(x, edge_index, edge_attr, W1a, b1a, W1b, b1b, g1, be1,
           W2a, b2a, W2b, b2b, g2, be2, Wo, bo):
    src = edge_index[0]
    dst = edge_index[1]
    z = jnp.zeros((NP, H), jnp.float32)
    ones = jnp.ones((CB, H), jnp.float32)

    cnt = _sc_count(dst, z, ones)

    A1, B1 = _node_proj(x, W1a, b1a)
    Ep1 = _edge_proj(edge_attr, W1a[2 * D:])
    Sp1 = _sc_edge(A1, B1, Ep1, dst, src, z)
    h1 = _post(Sp1, cnt, W1b, b1b, g1, be1)

    A2, B2 = _node_proj(h1, W2a, b2a)
    Ep2 = _edge_proj(edge_attr, W2a[2 * H:])
    Sp2 = _sc_edge(A2, B2, Ep2, dst, src, z)
    return _post_final(Sp2, cnt, W2b, b2b, g2, be2, Wo, bo)


# speedup vs baseline: 1.5116x; 1.5116x over previous
import functools

import jax
import jax.numpy as jnp
from jax import lax
from jax.experimental import pallas as pl
from jax.experimental.pallas import tpu as pltpu
from jax.experimental.pallas import tpu_sc as plsc

N = 10000
E = 320000
D = 128
DE = 16
H = 128
OUT = 4
EPS = 1e-5

L = 16
NC = 2
NS = 16
NW = NC * NS
EPW = E // NW
CB = 80
NCHUNK = EPW // CB
RPT = 632
NP = RPT * NS

BN = 2000
BE = 4000


def _node_proj_body(h_ref, w_ref, b_ref, a_ref, b_out_ref):
    h = h_ref[...]
    a_ref[...] = jnp.dot(h, w_ref[0:D, :], preferred_element_type=jnp.float32) + b_ref[...]
    b_out_ref[...] = jnp.dot(h, w_ref[D:2 * D, :], preferred_element_type=jnp.float32)


def _node_proj(h, W, b):
    return pl.pallas_call(
        _node_proj_body,
        grid=(N // BN,),
        in_specs=[
            pl.BlockSpec((BN, D), lambda i: (i, 0)),
            pl.BlockSpec((2 * D + DE, H), lambda i: (0, 0)),
            pl.BlockSpec((1, H), lambda i: (0, 0)),
        ],
        out_specs=[
            pl.BlockSpec((BN, H), lambda i: (i, 0)),
            pl.BlockSpec((BN, H), lambda i: (i, 0)),
        ],
        out_shape=[jax.ShapeDtypeStruct((N, H), jnp.float32)] * 2,
    )(h, W, b.reshape(1, H))


def _edge_proj_body(ea_ref, w_ref, o_ref):
    o_ref[...] = jnp.dot(ea_ref[...], w_ref[...], preferred_element_type=jnp.float32)


def _edge_proj(ea, We):
    return pl.pallas_call(
        _edge_proj_body,
        grid=(E // BE,),
        in_specs=[
            pl.BlockSpec((BE, DE), lambda i: (i, 0)),
            pl.BlockSpec((DE, H), lambda i: (0, 0)),
        ],
        out_specs=pl.BlockSpec((BE, H), lambda i: (i, 0)),
        out_shape=jax.ShapeDtypeStruct((E, H), jnp.float32),
    )(ea, We)


def _mean_bn_relu(sp_ref, cnt_ref, wb_ref, bb_ref, g_ref, be_ref):
    S = sp_ref[0, 0:N, :] + sp_ref[1, 0:N, :]
    cnt = cnt_ref[0, 0:N, 0:1] + cnt_ref[1, 0:N, 0:1]
    m = jnp.dot(S, wb_ref[...], preferred_element_type=jnp.float32)
    m = (m + cnt * bb_ref[...]) / jnp.maximum(cnt, 1.0)
    mu = jnp.mean(m, axis=0, keepdims=True)
    var = jnp.mean((m - mu) ** 2, axis=0, keepdims=True)
    return jnp.maximum((m - mu) * lax.rsqrt(var + EPS) * g_ref[...] + be_ref[...], 0.0)


def _post_body(sp_ref, cnt_ref, wb_ref, bb_ref, g_ref, be_ref, h_ref):
    h_ref[...] = _mean_bn_relu(sp_ref, cnt_ref, wb_ref, bb_ref, g_ref, be_ref)


def _post(Sp, cnt, Wb, bb, g, be):
    return pl.pallas_call(
        _post_body,
        out_shape=jax.ShapeDtypeStruct((N, H), jnp.float32),
    )(Sp, cnt, Wb, bb.reshape(1, H), g.reshape(1, H), be.reshape(1, H))


def _post_final_body(sp_ref, cnt_ref, wb_ref, bb_ref, g_ref, be_ref,
                     wo_ref, bo_ref, o_ref):
    h = _mean_bn_relu(sp_ref, cnt_ref, wb_ref, bb_ref, g_ref, be_ref)
    o_ref[...] = jnp.dot(h, wo_ref[...], preferred_element_type=jnp.float32) + bo_ref[...]


def _post_final(Sp, cnt, Wb, bb, g, be, Wo, bo):
    return pl.pallas_call(
        _post_final_body,
        out_shape=jax.ShapeDtypeStruct((N, OUT), jnp.float32),
    )(Sp, cnt, Wb, bb.reshape(1, H), g.reshape(1, H), be.reshape(1, H),
      Wo, bo.reshape(1, OUT))


def _sc_edge_body(a_hbm, b_hbm, ep_hbm, dst_hbm, src_hbm, z_hbm, out_hbm,
                  dsti, srci, arows, brows, eprov, msg, S, sema, semb):
    cid = lax.axis_index("c")
    sid = lax.axis_index("s")
    wid = sid * NC + cid
    base = wid * EPW

    pltpu.sync_copy(z_hbm.at[pl.ds(sid * RPT, RPT)], S.at[pl.ds(sid * RPT, RPT)])
    plsc.subcore_barrier()

    def chunk(i, c):
        off = base + i * CB
        pltpu.sync_copy(dst_hbm.at[pl.ds(off, CB)], dsti)
        pltpu.sync_copy(src_hbm.at[pl.ds(off, CB)], srci)
        ga = pltpu.async_copy(a_hbm.at[dsti], arows, sema)
        gb = pltpu.async_copy(b_hbm.at[srci], brows, semb)
        ge = pltpu.async_copy(ep_hbm.at[pl.ds(off, CB)], eprov, sema)
        ga.wait()
        gb.wait()
        ge.wait()

        def row(r, cc):
            for k in range(H // L):
                s = k * L
                v = arows[r, pl.ds(s, L)] + brows[r, pl.ds(s, L)] + eprov[r, pl.ds(s, L)]
                msg[r, pl.ds(s, L)] = jnp.maximum(v, 0.0)
            return cc

        lax.fori_loop(0, CB, row, 0)
        pltpu.sync_copy(msg, S.at[dsti], add=True)
        return c

    lax.fori_loop(0, NCHUNK, chunk, 0)
    plsc.subcore_barrier()

    pltpu.sync_copy(S.at[pl.ds(sid * RPT, RPT)],
                    out_hbm.at[cid, pl.ds(sid * RPT, RPT)])


_sc_edge = functools.partial(
    pl.kernel,
    mesh=plsc.VectorSubcoreMesh(core_axis_name="c", subcore_axis_name="s",
                                num_cores=NC, num_subcores=NS),
    out_type=jax.ShapeDtypeStruct((NC, NP, H), jnp.float32),
    scratch_types=[
        pltpu.VMEM((CB,), jnp.int32),
        pltpu.VMEM((CB,), jnp.int32),
        pltpu.VMEM((CB, H), jnp.float32),
        pltpu.VMEM((CB, H), jnp.float32),
        pltpu.VMEM((CB, H), jnp.float32),
        pltpu.VMEM((CB, H), jnp.float32),
        pltpu.VMEM_SHARED((NP, H), jnp.float32),
        pltpu.SemaphoreType.DMA,
        pltpu.SemaphoreType.DMA,
    ],
)(_sc_edge_body)


def _sc_count_body(dst_hbm, z_hbm, ones_hbm, out_hbm, dsti, ones_v, S):
    cid = lax.axis_index("c")
    sid = lax.axis_index("s")
    wid = sid * NC + cid

    pltpu.sync_copy(z_hbm.at[pl.ds(sid * RPT, RPT)], S.at[pl.ds(sid * RPT, RPT)])
    pltpu.sync_copy(ones_hbm, ones_v)
    plsc.subcore_barrier()

    base = wid * EPW

    def chunk(i, c):
        off = base + i * CB
        pltpu.sync_copy(dst_hbm.at[pl.ds(off, CB)], dsti)
        pltpu.sync_copy(ones_v, S.at[dsti], add=True)
        return c

    lax.fori_loop(0, NCHUNK, chunk, 0)
    plsc.subcore_barrier()

    pltpu.sync_copy(S.at[pl.ds(sid * RPT, RPT)],
                    out_hbm.at[cid, pl.ds(sid * RPT, RPT)])


_sc_count = functools.partial(
    pl.kernel,
    mesh=plsc.VectorSubcoreMesh(core_axis_name="c", subcore_axis_name="s",
                                num_cores=NC, num_subcores=NS),
    out_type=jax.ShapeDtypeStruct((NC, NP, H), jnp.float32),
    scratch_types=[
        pltpu.VMEM((CB,), jnp.int32),
        pltpu.VMEM((CB, H), jnp.float32),
        pltpu.VMEM_SHARED((NP, H), jnp.float32),
    ],
)(_sc_count_body)


def kernel(x, edge_index, edge_attr, W1a, b1a, W1b, b1b, g1, be1,
           W2a, b2a, W2b, b2b, g2, be2, Wo, bo):
    src = edge_index[0]
    dst = edge_index[1]
    z = jnp.zeros((NP, H), jnp.float32)
    ones = jnp.ones((CB, H), jnp.float32)

    cnt = _sc_count(dst, z, ones)

    A1, B1 = _node_proj(x, W1a, b1a)
    Ep1 = _edge_proj(edge_attr, W1a[2 * D:])
    Sp1 = _sc_edge(A1, B1, Ep1, dst, src, z)
    h1 = _post(Sp1, cnt, W1b, b1b, g1, be1)

    A2, B2 = _node_proj(h1, W2a, b2a)
    Ep2 = _edge_proj(edge_attr, W2a[2 * H:])
    Sp2 = _sc_edge(A2, B2, Ep2, dst, src, z)
    return _post_final(Sp2, cnt, W2b, b2b, g2, be2, Wo, bo)


# speedup vs baseline: 1.5226x; 1.0073x over previous
import functools

import jax
import jax.numpy as jnp
from jax import lax
from jax.experimental import pallas as pl
from jax.experimental.pallas import tpu as pltpu
from jax.experimental.pallas import tpu_sc as plsc

N = 10000
E = 320000
D = 128
DE = 16
H = 128
OUT = 4
EPS = 1e-5

L = 16
NC = 2
NS = 16
NW = NC * NS
EPW = E // NW
CB = 80
NCHUNK = EPW // CB
RPT = 632
NP = RPT * NS

BN = 2000
BE = 4000


def _node_proj_body(h_ref, w_ref, b_ref, a_ref, b_out_ref):
    h = h_ref[...]
    a_ref[...] = jnp.dot(h, w_ref[0:D, :], preferred_element_type=jnp.float32) + b_ref[...]
    b_out_ref[...] = jnp.dot(h, w_ref[D:2 * D, :], preferred_element_type=jnp.float32)


def _node_proj(h, W, b):
    return pl.pallas_call(
        _node_proj_body,
        grid=(N // BN,),
        in_specs=[
            pl.BlockSpec((BN, D), lambda i: (i, 0)),
            pl.BlockSpec((2 * D + DE, H), lambda i: (0, 0)),
            pl.BlockSpec((1, H), lambda i: (0, 0)),
        ],
        out_specs=[
            pl.BlockSpec((BN, H), lambda i: (i, 0)),
            pl.BlockSpec((BN, H), lambda i: (i, 0)),
        ],
        out_shape=[jax.ShapeDtypeStruct((N, H), jnp.float32)] * 2,
    )(h, W, b.reshape(1, H))


def _edge_proj_body(ea_ref, w_ref, o_ref):
    o_ref[...] = jnp.dot(ea_ref[...], w_ref[...], preferred_element_type=jnp.float32)


def _edge_proj(ea, We):
    return pl.pallas_call(
        _edge_proj_body,
        grid=(E // BE,),
        in_specs=[
            pl.BlockSpec((BE, DE), lambda i: (i, 0)),
            pl.BlockSpec((DE, H), lambda i: (0, 0)),
        ],
        out_specs=pl.BlockSpec((BE, H), lambda i: (i, 0)),
        out_shape=jax.ShapeDtypeStruct((E, H), jnp.float32),
    )(ea, We)


def _mean_bn_relu(sp_ref, cnt_ref, wb_ref, bb_ref, g_ref, be_ref):
    S = sp_ref[0, 0:N, :] + sp_ref[1, 0:N, :]
    cnt = cnt_ref[0, 0:N, 0:1] + cnt_ref[1, 0:N, 0:1]
    m = jnp.dot(S, wb_ref[...], preferred_element_type=jnp.float32)
    m = (m + cnt * bb_ref[...]) / jnp.maximum(cnt, 1.0)
    mu = jnp.mean(m, axis=0, keepdims=True)
    var = jnp.mean((m - mu) ** 2, axis=0, keepdims=True)
    return jnp.maximum((m - mu) * lax.rsqrt(var + EPS) * g_ref[...] + be_ref[...], 0.0)


def _post_body(sp_ref, cnt_ref, wb_ref, bb_ref, g_ref, be_ref, h_ref):
    h_ref[...] = _mean_bn_relu(sp_ref, cnt_ref, wb_ref, bb_ref, g_ref, be_ref)


def _post(Sp, cnt, Wb, bb, g, be):
    return pl.pallas_call(
        _post_body,
        out_shape=jax.ShapeDtypeStruct((N, H), jnp.float32),
    )(Sp, cnt, Wb, bb.reshape(1, H), g.reshape(1, H), be.reshape(1, H))


def _post_final_body(sp_ref, cnt_ref, wb_ref, bb_ref, g_ref, be_ref,
                     wo_ref, bo_ref, o_ref):
    h = _mean_bn_relu(sp_ref, cnt_ref, wb_ref, bb_ref, g_ref, be_ref)
    o_ref[...] = jnp.dot(h, wo_ref[...], preferred_element_type=jnp.float32) + bo_ref[...]


def _post_final(Sp, cnt, Wb, bb, g, be, Wo, bo):
    return pl.pallas_call(
        _post_final_body,
        out_shape=jax.ShapeDtypeStruct((N, OUT), jnp.float32),
    )(Sp, cnt, Wb, bb.reshape(1, H), g.reshape(1, H), be.reshape(1, H),
      Wo, bo.reshape(1, OUT))


def _sc_edge_body(a_hbm, b_hbm, ep_hbm, dst_hbm, src_hbm, z_hbm, out_hbm,
                  dsti, srci, arows, brows, eprov, msg, S, sema, semb):
    cid = lax.axis_index("c")
    sid = lax.axis_index("s")
    wid = sid * NC + cid
    base = wid * EPW

    pltpu.sync_copy(z_hbm.at[pl.ds(sid * RPT, RPT)], S.at[pl.ds(sid * RPT, RPT)])
    plsc.subcore_barrier()

    def chunk(i, c):
        off = base + i * CB
        pltpu.sync_copy(dst_hbm.at[pl.ds(off, CB)], dsti)
        pltpu.sync_copy(src_hbm.at[pl.ds(off, CB)], srci)
        ga = pltpu.async_copy(a_hbm.at[dsti], arows, sema)
        gb = pltpu.async_copy(b_hbm.at[srci], brows, semb)
        pltpu.sync_copy(ep_hbm.at[pl.ds(off, CB)], eprov)
        ga.wait()
        gb.wait()

        def row(r, cc):
            for k in range(H // L):
                s = k * L
                v = arows[r, pl.ds(s, L)] + brows[r, pl.ds(s, L)] + eprov[r, pl.ds(s, L)]
                msg[r, pl.ds(s, L)] = jnp.maximum(v, 0.0)
            return cc

        lax.fori_loop(0, CB, row, 0)
        pltpu.sync_copy(msg, S.at[dsti], add=True)
        return c

    lax.fori_loop(0, NCHUNK, chunk, 0)
    plsc.subcore_barrier()

    pltpu.sync_copy(S.at[pl.ds(sid * RPT, RPT)],
                    out_hbm.at[cid, pl.ds(sid * RPT, RPT)])


_sc_edge = functools.partial(
    pl.kernel,
    mesh=plsc.VectorSubcoreMesh(core_axis_name="c", subcore_axis_name="s",
                                num_cores=NC, num_subcores=NS),
    out_type=jax.ShapeDtypeStruct((NC, NP, H), jnp.float32),
    scratch_types=[
        pltpu.VMEM((CB,), jnp.int32),
        pltpu.VMEM((CB,), jnp.int32),
        pltpu.VMEM((CB, H), jnp.float32),
        pltpu.VMEM((CB, H), jnp.float32),
        pltpu.VMEM((CB, H), jnp.float32),
        pltpu.VMEM((CB, H), jnp.float32),
        pltpu.VMEM_SHARED((NP, H), jnp.float32),
        pltpu.SemaphoreType.DMA,
        pltpu.SemaphoreType.DMA,
    ],
)(_sc_edge_body)


def _sc_count_body(dst_hbm, z_hbm, ones_hbm, out_hbm, dsti, ones_v, S):
    cid = lax.axis_index("c")
    sid = lax.axis_index("s")
    wid = sid * NC + cid

    pltpu.sync_copy(z_hbm.at[pl.ds(sid * RPT, RPT)], S.at[pl.ds(sid * RPT, RPT)])
    pltpu.sync_copy(ones_hbm, ones_v)
    plsc.subcore_barrier()

    base = wid * EPW

    def chunk(i, c):
        off = base + i * CB
        pltpu.sync_copy(dst_hbm.at[pl.ds(off, CB)], dsti)
        pltpu.sync_copy(ones_v, S.at[dsti], add=True)
        return c

    lax.fori_loop(0, NCHUNK, chunk, 0)
    plsc.subcore_barrier()

    pltpu.sync_copy(S.at[pl.ds(sid * RPT, RPT)],
                    out_hbm.at[cid, pl.ds(sid * RPT, RPT)])


_sc_count = functools.partial(
    pl.kernel,
    mesh=plsc.VectorSubcoreMesh(core_axis_name="c", subcore_axis_name="s",
                                num_cores=NC, num_subcores=NS),
    out_type=jax.ShapeDtypeStruct((NC, NP, H), jnp.float32),
    scratch_types=[
        pltpu.VMEM((CB,), jnp.int32),
        pltpu.VMEM((CB, H), jnp.float32),
        pltpu.VMEM_SHARED((NP, H), jnp.float32),
    ],
)(_sc_count_body)


def kernel(x, edge_index, edge_attr, W1a, b1a, W1b, b1b, g1, be1,
           W2a, b2a, W2b, b2b, g2, be2, Wo, bo):
    src = edge_index[0]
    dst = edge_index[1]
    z = jnp.zeros((NP, H), jnp.float32)
    ones = jnp.ones((CB, H), jnp.float32)

    cnt = _sc_count(dst, z, ones)

    A1, B1 = _node_proj(x, W1a, b1a)
    Ep1 = _edge_proj(edge_attr, W1a[2 * D:])
    Sp1 = _sc_edge(A1, B1, Ep1, dst, src, z)
    h1 = _post(Sp1, cnt, W1b, b1b, g1, be1)

    A2, B2 = _node_proj(h1, W2a, b2a)
    Ep2 = _edge_proj(edge_attr, W2a[2 * H:])
    Sp2 = _sc_edge(A2, B2, Ep2, dst, src, z)
    return _post_final(Sp2, cnt, W2b, b2b, g2, be2, Wo, bo)


# speedup vs baseline: 1.5303x; 1.0051x over previous
import functools

import jax
import jax.numpy as jnp
from jax import lax
from jax.experimental import pallas as pl
from jax.experimental.pallas import tpu as pltpu
from jax.experimental.pallas import tpu_sc as plsc

N = 10000
E = 320000
D = 128
DE = 16
H = 128
OUT = 4
EPS = 1e-5

L = 16
NC = 2
NS = 16
NW = NC * NS
EPW = E // NW
CB = 80
NCHUNK = EPW // CB
RPT = 632
NP = RPT * NS

BN = 2000
BE = 4000


def _node_proj_body(h_ref, w_ref, b_ref, a_ref, b_out_ref):
    h = h_ref[...]
    a_ref[...] = jnp.dot(h, w_ref[0:D, :], preferred_element_type=jnp.float32) + b_ref[...]
    b_out_ref[...] = jnp.dot(h, w_ref[D:2 * D, :], preferred_element_type=jnp.float32)


def _node_proj(h, W, b):
    return pl.pallas_call(
        _node_proj_body,
        grid=(N // BN,),
        in_specs=[
            pl.BlockSpec((BN, D), lambda i: (i, 0)),
            pl.BlockSpec((2 * D + DE, H), lambda i: (0, 0)),
            pl.BlockSpec((1, H), lambda i: (0, 0)),
        ],
        out_specs=[
            pl.BlockSpec((BN, H), lambda i: (i, 0)),
            pl.BlockSpec((BN, H), lambda i: (i, 0)),
        ],
        out_shape=[jax.ShapeDtypeStruct((N, H), jnp.float32)] * 2,
    )(h, W, b.reshape(1, H))


def _edge_proj_body(ea_ref, w_ref, o_ref):
    o_ref[...] = jnp.dot(ea_ref[...], w_ref[...], preferred_element_type=jnp.float32)


def _edge_proj(ea, We):
    return pl.pallas_call(
        _edge_proj_body,
        grid=(E // BE,),
        in_specs=[
            pl.BlockSpec((BE, DE), lambda i: (i, 0)),
            pl.BlockSpec((DE, H), lambda i: (0, 0)),
        ],
        out_specs=pl.BlockSpec((BE, H), lambda i: (i, 0)),
        out_shape=jax.ShapeDtypeStruct((E, H), jnp.float32),
    )(ea, We)


def _mean_bn_relu(sp_ref, cnt_ref, wb_ref, bb_ref, g_ref, be_ref):
    S = sp_ref[0, 0:N, :] + sp_ref[1, 0:N, :]
    cnt = cnt_ref[0, 0:N, 0:1] + cnt_ref[1, 0:N, 0:1]
    m = jnp.dot(S, wb_ref[...], preferred_element_type=jnp.float32)
    m = (m + cnt * bb_ref[...]) / jnp.maximum(cnt, 1.0)
    mu = jnp.mean(m, axis=0, keepdims=True)
    var = jnp.mean((m - mu) ** 2, axis=0, keepdims=True)
    return jnp.maximum((m - mu) * lax.rsqrt(var + EPS) * g_ref[...] + be_ref[...], 0.0)


def _post_body(sp_ref, cnt_ref, wb_ref, bb_ref, g_ref, be_ref, h_ref):
    h_ref[...] = _mean_bn_relu(sp_ref, cnt_ref, wb_ref, bb_ref, g_ref, be_ref)


def _post(Sp, cnt, Wb, bb, g, be):
    return pl.pallas_call(
        _post_body,
        out_shape=jax.ShapeDtypeStruct((N, H), jnp.float32),
    )(Sp, cnt, Wb, bb.reshape(1, H), g.reshape(1, H), be.reshape(1, H))


def _post_final_body(sp_ref, cnt_ref, wb_ref, bb_ref, g_ref, be_ref,
                     wo_ref, bo_ref, o_ref):
    h = _mean_bn_relu(sp_ref, cnt_ref, wb_ref, bb_ref, g_ref, be_ref)
    o_ref[...] = jnp.dot(h, wo_ref[...], preferred_element_type=jnp.float32) + bo_ref[...]


def _post_final(Sp, cnt, Wb, bb, g, be, Wo, bo):
    return pl.pallas_call(
        _post_final_body,
        out_shape=jax.ShapeDtypeStruct((N, OUT), jnp.float32),
    )(Sp, cnt, Wb, bb.reshape(1, H), g.reshape(1, H), be.reshape(1, H),
      Wo, bo.reshape(1, OUT))


def _sc_edge_body(a_hbm, b_hbm, ep_hbm, dst_hbm, src_hbm, z_hbm, out_hbm,
                  dsti, srci, arows, brows, eprov, msg, S, sema, semb):
    cid = lax.axis_index("c")
    sid = lax.axis_index("s")
    wid = sid * NC + cid
    base = wid * EPW

    pltpu.sync_copy(z_hbm.at[pl.ds(sid * RPT, RPT)], S.at[pl.ds(sid * RPT, RPT)])
    plsc.subcore_barrier()

    def chunk(i, c):
        off = base + i * CB
        pltpu.sync_copy(dst_hbm.at[pl.ds(off, CB)], dsti)
        pltpu.sync_copy(src_hbm.at[pl.ds(off, CB)], srci)
        ga = pltpu.async_copy(a_hbm.at[dsti], arows, sema)
        gb = pltpu.async_copy(b_hbm.at[srci], brows, semb)
        pltpu.sync_copy(ep_hbm.at[pl.ds(off, CB)], eprov)
        ga.wait()
        gb.wait()

        def row(r, cc):
            for k in range(H // L):
                s = k * L
                v = arows[r, pl.ds(s, L)] + brows[r, pl.ds(s, L)] + eprov[r, pl.ds(s, L)]
                msg[r, pl.ds(s, L)] = jnp.maximum(v, 0.0)
            return cc

        lax.fori_loop(0, CB, row, 0)
        pltpu.sync_copy(msg, S.at[dsti], add=True)
        return c

    lax.fori_loop(0, NCHUNK, chunk, 0)
    plsc.subcore_barrier()

    pltpu.sync_copy(S.at[pl.ds(sid * RPT, RPT)],
                    out_hbm.at[cid, pl.ds(sid * RPT, RPT)])


_sc_edge = functools.partial(
    pl.kernel,
    mesh=plsc.VectorSubcoreMesh(core_axis_name="c", subcore_axis_name="s",
                                num_cores=NC, num_subcores=NS),
    out_type=jax.ShapeDtypeStruct((NC, NP, H), jnp.float32),
    scratch_types=[
        pltpu.VMEM((CB,), jnp.int32),
        pltpu.VMEM((CB,), jnp.int32),
        pltpu.VMEM((CB, H), jnp.float32),
        pltpu.VMEM((CB, H), jnp.float32),
        pltpu.VMEM((CB, H), jnp.float32),
        pltpu.VMEM((CB, H), jnp.float32),
        pltpu.VMEM_SHARED((NP, H), jnp.float32),
        pltpu.SemaphoreType.DMA,
        pltpu.SemaphoreType.DMA,
    ],
)(_sc_edge_body)


def _sc_edge_cnt_body(a_hbm, b_hbm, ep_hbm, dst_hbm, src_hbm, z_hbm, ones_hbm,
                      out_hbm, cnt_hbm,
                      dsti, srci, arows, brows, eprov, msg, S, sema, semb):
    cid = lax.axis_index("c")
    sid = lax.axis_index("s")
    wid = sid * NC + cid
    base = wid * EPW

    pltpu.sync_copy(z_hbm.at[pl.ds(sid * RPT, RPT)], S.at[pl.ds(sid * RPT, RPT)])
    pltpu.sync_copy(ones_hbm, msg)
    plsc.subcore_barrier()

    def cchunk(i, c):
        off = base + i * CB
        pltpu.sync_copy(dst_hbm.at[pl.ds(off, CB)], dsti)
        pltpu.sync_copy(msg, S.at[dsti], add=True)
        return c

    lax.fori_loop(0, NCHUNK, cchunk, 0)
    plsc.subcore_barrier()
    pltpu.sync_copy(S.at[pl.ds(sid * RPT, RPT)],
                    cnt_hbm.at[cid, pl.ds(sid * RPT, RPT)])
    plsc.subcore_barrier()

    pltpu.sync_copy(z_hbm.at[pl.ds(sid * RPT, RPT)], S.at[pl.ds(sid * RPT, RPT)])
    plsc.subcore_barrier()

    def chunk(i, c):
        off = base + i * CB
        pltpu.sync_copy(dst_hbm.at[pl.ds(off, CB)], dsti)
        pltpu.sync_copy(src_hbm.at[pl.ds(off, CB)], srci)
        ga = pltpu.async_copy(a_hbm.at[dsti], arows, sema)
        gb = pltpu.async_copy(b_hbm.at[srci], brows, semb)
        pltpu.sync_copy(ep_hbm.at[pl.ds(off, CB)], eprov)
        ga.wait()
        gb.wait()

        def row(r, cc):
            for k in range(H // L):
                s = k * L
                v = arows[r, pl.ds(s, L)] + brows[r, pl.ds(s, L)] + eprov[r, pl.ds(s, L)]
                msg[r, pl.ds(s, L)] = jnp.maximum(v, 0.0)
            return cc

        lax.fori_loop(0, CB, row, 0)
        pltpu.sync_copy(msg, S.at[dsti], add=True)
        return c

    lax.fori_loop(0, NCHUNK, chunk, 0)
    plsc.subcore_barrier()

    pltpu.sync_copy(S.at[pl.ds(sid * RPT, RPT)],
                    out_hbm.at[cid, pl.ds(sid * RPT, RPT)])


_sc_edge_cnt = functools.partial(
    pl.kernel,
    mesh=plsc.VectorSubcoreMesh(core_axis_name="c", subcore_axis_name="s",
                                num_cores=NC, num_subcores=NS),
    out_type=(jax.ShapeDtypeStruct((NC, NP, H), jnp.float32),
              jax.ShapeDtypeStruct((NC, NP, H), jnp.float32)),
    scratch_types=[
        pltpu.VMEM((CB,), jnp.int32),
        pltpu.VMEM((CB,), jnp.int32),
        pltpu.VMEM((CB, H), jnp.float32),
        pltpu.VMEM((CB, H), jnp.float32),
        pltpu.VMEM((CB, H), jnp.float32),
        pltpu.VMEM((CB, H), jnp.float32),
        pltpu.VMEM_SHARED((NP, H), jnp.float32),
        pltpu.SemaphoreType.DMA,
        pltpu.SemaphoreType.DMA,
    ],
)(_sc_edge_cnt_body)


def _sc_count_body(dst_hbm, z_hbm, ones_hbm, out_hbm, dsti, ones_v, S):
    cid = lax.axis_index("c")
    sid = lax.axis_index("s")
    wid = sid * NC + cid

    pltpu.sync_copy(z_hbm.at[pl.ds(sid * RPT, RPT)], S.at[pl.ds(sid * RPT, RPT)])
    pltpu.sync_copy(ones_hbm, ones_v)
    plsc.subcore_barrier()

    base = wid * EPW

    def chunk(i, c):
        off = base + i * CB
        pltpu.sync_copy(dst_hbm.at[pl.ds(off, CB)], dsti)
        pltpu.sync_copy(ones_v, S.at[dsti], add=True)
        return c

    lax.fori_loop(0, NCHUNK, chunk, 0)
    plsc.subcore_barrier()

    pltpu.sync_copy(S.at[pl.ds(sid * RPT, RPT)],
                    out_hbm.at[cid, pl.ds(sid * RPT, RPT)])


_sc_count = functools.partial(
    pl.kernel,
    mesh=plsc.VectorSubcoreMesh(core_axis_name="c", subcore_axis_name="s",
                                num_cores=NC, num_subcores=NS),
    out_type=jax.ShapeDtypeStruct((NC, NP, H), jnp.float32),
    scratch_types=[
        pltpu.VMEM((CB,), jnp.int32),
        pltpu.VMEM((CB, H), jnp.float32),
        pltpu.VMEM_SHARED((NP, H), jnp.float32),
    ],
)(_sc_count_body)


def kernel(x, edge_index, edge_attr, W1a, b1a, W1b, b1b, g1, be1,
           W2a, b2a, W2b, b2b, g2, be2, Wo, bo):
    src = edge_index[0]
    dst = edge_index[1]
    z = jnp.zeros((NP, H), jnp.float32)
    ones = jnp.ones((CB, H), jnp.float32)

    A1, B1 = _node_proj(x, W1a, b1a)
    Ep1 = _edge_proj(edge_attr, W1a[2 * D:])
    Sp1, cnt = _sc_edge_cnt(A1, B1, Ep1, dst, src, z, ones)
    h1 = _post(Sp1, cnt, W1b, b1b, g1, be1)

    A2, B2 = _node_proj(h1, W2a, b2a)
    Ep2 = _edge_proj(edge_attr, W2a[2 * H:])
    Sp2 = _sc_edge(A2, B2, Ep2, dst, src, z)
    return _post_final(Sp2, cnt, W2b, b2b, g2, be2, Wo, bo)


# speedup vs baseline: 1.5304x; 1.0001x over previous
import functools

import jax
import jax.numpy as jnp
from jax import lax
from jax.experimental import pallas as pl
from jax.experimental.pallas import tpu as pltpu
from jax.experimental.pallas import tpu_sc as plsc

N = 10000
E = 320000
D = 128
DE = 16
H = 128
OUT = 4
EPS = 1e-5

L = 16
NC = 2
NS = 16
NW = NC * NS
EPW = E // NW
CB = 80
NCHUNK = EPW // CB
RPT = 632
NP = RPT * NS

BN = 2000
BE = 4000


def _node_proj_body(h_ref, w_ref, b_ref, a_ref, b_out_ref):
    h = h_ref[...]
    a_ref[...] = jnp.dot(h, w_ref[0:D, :], preferred_element_type=jnp.float32) + b_ref[...]
    b_out_ref[...] = jnp.dot(h, w_ref[D:2 * D, :], preferred_element_type=jnp.float32)


def _node_proj(h, W, b):
    return pl.pallas_call(
        _node_proj_body,
        grid=(N // BN,),
        in_specs=[
            pl.BlockSpec((BN, D), lambda i: (i, 0)),
            pl.BlockSpec((2 * D + DE, H), lambda i: (0, 0)),
            pl.BlockSpec((1, H), lambda i: (0, 0)),
        ],
        out_specs=[
            pl.BlockSpec((BN, H), lambda i: (i, 0)),
            pl.BlockSpec((BN, H), lambda i: (i, 0)),
        ],
        out_shape=[jax.ShapeDtypeStruct((N, H), jnp.float32)] * 2,
    )(h, W, b.reshape(1, H))


def _edge_proj_body(ea_ref, w_ref, o_ref):
    o_ref[...] = jnp.dot(ea_ref[...], w_ref[...], preferred_element_type=jnp.float32)


def _edge_proj(ea, We):
    return pl.pallas_call(
        _edge_proj_body,
        grid=(E // BE,),
        in_specs=[
            pl.BlockSpec((BE, DE), lambda i: (i, 0)),
            pl.BlockSpec((DE, H), lambda i: (0, 0)),
        ],
        out_specs=pl.BlockSpec((BE, H), lambda i: (i, 0)),
        out_shape=jax.ShapeDtypeStruct((E, H), jnp.float32),
    )(ea, We)


def _mean_bn_relu(sp_ref, cnt_ref, wb_ref, bb_ref, g_ref, be_ref):
    S = sp_ref[0, 0:N, :] + sp_ref[1, 0:N, :]
    cnt = cnt_ref[0, 0:N, 0:1] + cnt_ref[1, 0:N, 0:1]
    m = jnp.dot(S, wb_ref[...], preferred_element_type=jnp.float32)
    m = (m + cnt * bb_ref[...]) / jnp.maximum(cnt, 1.0)
    mu = jnp.mean(m, axis=0, keepdims=True)
    var = jnp.mean((m - mu) ** 2, axis=0, keepdims=True)
    return jnp.maximum((m - mu) * lax.rsqrt(var + EPS) * g_ref[...] + be_ref[...], 0.0)


def _post_body(sp_ref, cnt_ref, wb_ref, bb_ref, g_ref, be_ref, h_ref):
    h_ref[...] = _mean_bn_relu(sp_ref, cnt_ref, wb_ref, bb_ref, g_ref, be_ref)


def _post(Sp, cnt, Wb, bb, g, be):
    return pl.pallas_call(
        _post_body,
        out_shape=jax.ShapeDtypeStruct((N, H), jnp.float32),
    )(Sp, cnt, Wb, bb.reshape(1, H), g.reshape(1, H), be.reshape(1, H))


def _post_final_body(sp_ref, cnt_ref, wb_ref, bb_ref, g_ref, be_ref,
                     wo_ref, bo_ref, o_ref):
    h = _mean_bn_relu(sp_ref, cnt_ref, wb_ref, bb_ref, g_ref, be_ref)
    o_ref[...] = jnp.dot(h, wo_ref[...], preferred_element_type=jnp.float32) + bo_ref[...]


def _post_final(Sp, cnt, Wb, bb, g, be, Wo, bo):
    return pl.pallas_call(
        _post_final_body,
        out_shape=jax.ShapeDtypeStruct((N, OUT), jnp.float32),
    )(Sp, cnt, Wb, bb.reshape(1, H), g.reshape(1, H), be.reshape(1, H),
      Wo, bo.reshape(1, OUT))


def _sc_edge_body(a_hbm, b_hbm, ep_hbm, dst_hbm, src_hbm, z_hbm, out_hbm,
                  dsti, srci, arows, brows, eprov, msg, S, sema, semb):
    cid = lax.axis_index("c")
    sid = lax.axis_index("s")
    wid = sid * NC + cid
    base = wid * EPW

    pltpu.sync_copy(z_hbm.at[pl.ds(sid * RPT, RPT)], S.at[pl.ds(sid * RPT, RPT)])
    plsc.subcore_barrier()

    def chunk(i, c):
        off = base + i * CB
        pltpu.sync_copy(dst_hbm.at[pl.ds(off, CB)], dsti)
        pltpu.sync_copy(src_hbm.at[pl.ds(off, CB)], srci)
        ga = pltpu.async_copy(a_hbm.at[dsti], arows, sema)
        gb = pltpu.async_copy(b_hbm.at[srci], brows, semb)
        pltpu.sync_copy(ep_hbm.at[pl.ds(off, CB)], eprov)
        ga.wait()
        gb.wait()

        def row(r, cc):
            for k in range(H // L):
                s = k * L
                v = arows[r, pl.ds(s, L)] + brows[r, pl.ds(s, L)] + eprov[r, pl.ds(s, L)]
                msg[r, pl.ds(s, L)] = jnp.maximum(v, 0.0)
            return cc

        lax.fori_loop(0, CB, row, 0)
        pltpu.sync_copy(msg, S.at[dsti], add=True)
        return c

    lax.fori_loop(0, NCHUNK, chunk, 0)
    plsc.subcore_barrier()

    pltpu.sync_copy(S.at[pl.ds(sid * RPT, RPT)],
                    out_hbm.at[cid, pl.ds(sid * RPT, RPT)])


_sc_edge = functools.partial(
    pl.kernel,
    mesh=plsc.VectorSubcoreMesh(core_axis_name="c", subcore_axis_name="s",
                                num_cores=NC, num_subcores=NS),
    out_type=jax.ShapeDtypeStruct((NC, NP, H), jnp.float32),
    scratch_types=[
        pltpu.VMEM((CB,), jnp.int32),
        pltpu.VMEM((CB,), jnp.int32),
        pltpu.VMEM((CB, H), jnp.float32),
        pltpu.VMEM((CB, H), jnp.float32),
        pltpu.VMEM((CB, H), jnp.float32),
        pltpu.VMEM((CB, H), jnp.float32),
        pltpu.VMEM_SHARED((NP, H), jnp.float32),
        pltpu.SemaphoreType.DMA,
        pltpu.SemaphoreType.DMA,
    ],
)(_sc_edge_body)


def _sc_edge_cnt_body(a_hbm, b_hbm, ep_hbm, dst_hbm, src_hbm, z_hbm, ones_hbm,
                      out_hbm, cnt_hbm,
                      dsti, srci, arows, brows, eprov, msg, S, sema, semb):
    cid = lax.axis_index("c")
    sid = lax.axis_index("s")
    wid = sid * NC + cid
    base = wid * EPW

    pltpu.sync_copy(z_hbm.at[pl.ds(sid * RPT, RPT)], S.at[pl.ds(sid * RPT, RPT)])
    pltpu.sync_copy(ones_hbm, msg)
    plsc.subcore_barrier()

    def cchunk(i, c):
        off = base + i * CB
        pltpu.sync_copy(dst_hbm.at[pl.ds(off, CB)], dsti)
        pltpu.sync_copy(msg, S.at[dsti], add=True)
        return c

    lax.fori_loop(0, NCHUNK, cchunk, 0)
    plsc.subcore_barrier()
    pltpu.sync_copy(S.at[pl.ds(sid * RPT, RPT)],
                    cnt_hbm.at[cid, pl.ds(sid * RPT, RPT)])
    plsc.subcore_barrier()

    pltpu.sync_copy(z_hbm.at[pl.ds(sid * RPT, RPT)], S.at[pl.ds(sid * RPT, RPT)])
    plsc.subcore_barrier()

    def chunk(i, c):
        off = base + i * CB
        pltpu.sync_copy(dst_hbm.at[pl.ds(off, CB)], dsti)
        pltpu.sync_copy(src_hbm.at[pl.ds(off, CB)], srci)
        ga = pltpu.async_copy(a_hbm.at[dsti], arows, sema)
        gb = pltpu.async_copy(b_hbm.at[srci], brows, semb)
        pltpu.sync_copy(ep_hbm.at[pl.ds(off, CB)], eprov)
        ga.wait()
        gb.wait()

        def row(r, cc):
            for k in range(H // L):
                s = k * L
                v = arows[r, pl.ds(s, L)] + brows[r, pl.ds(s, L)] + eprov[r, pl.ds(s, L)]
                msg[r, pl.ds(s, L)] = jnp.maximum(v, 0.0)
            return cc

        lax.fori_loop(0, CB, row, 0)
        pltpu.sync_copy(msg, S.at[dsti], add=True)
        return c

    lax.fori_loop(0, NCHUNK, chunk, 0)
    plsc.subcore_barrier()

    pltpu.sync_copy(S.at[pl.ds(sid * RPT, RPT)],
                    out_hbm.at[cid, pl.ds(sid * RPT, RPT)])


_sc_edge_cnt = functools.partial(
    pl.kernel,
    mesh=plsc.VectorSubcoreMesh(core_axis_name="c", subcore_axis_name="s",
                                num_cores=NC, num_subcores=NS),
    out_type=(jax.ShapeDtypeStruct((NC, NP, H), jnp.float32),
              jax.ShapeDtypeStruct((NC, NP, H), jnp.float32)),
    scratch_types=[
        pltpu.VMEM((CB,), jnp.int32),
        pltpu.VMEM((CB,), jnp.int32),
        pltpu.VMEM((CB, H), jnp.float32),
        pltpu.VMEM((CB, H), jnp.float32),
        pltpu.VMEM((CB, H), jnp.float32),
        pltpu.VMEM((CB, H), jnp.float32),
        pltpu.VMEM_SHARED((NP, H), jnp.float32),
        pltpu.SemaphoreType.DMA,
        pltpu.SemaphoreType.DMA,
    ],
)(_sc_edge_cnt_body)


def kernel(x, edge_index, edge_attr, W1a, b1a, W1b, b1b, g1, be1,
           W2a, b2a, W2b, b2b, g2, be2, Wo, bo):
    src = edge_index[0]
    dst = edge_index[1]
    z = jnp.zeros((NP, H), jnp.float32)
    ones = jnp.ones((CB, H), jnp.float32)

    A1, B1 = _node_proj(x, W1a, b1a)
    Ep1 = _edge_proj(edge_attr, W1a[2 * D:])
    Sp1, cnt = _sc_edge_cnt(A1, B1, Ep1, dst, src, z, ones)
    h1 = _post(Sp1, cnt, W1b, b1b, g1, be1)

    A2, B2 = _node_proj(h1, W2a, b2a)
    Ep2 = _edge_proj(edge_attr, W2a[2 * H:])
    Sp2 = _sc_edge(A2, B2, Ep2, dst, src, z)
    return _post_final(Sp2, cnt, W2b, b2b, g2, be2, Wo, bo)
